# Initial kernel scaffold; baseline (speedup 1.0000x reference)
import jax
import jax.numpy as jnp
from jax.experimental import pallas as pl


def kernel(x, edge_index, edge_attr, batch, target_idx, Wp, bp, eps, We, be, W1, b1, W2, b2, gamma, beta, Temb, Wh1, bh1, Wh2, bh2, Wh3, bh3):
    raise NotImplementedError("write your pallas kernel here")


# speedup vs baseline: 1.0139x; 1.0139x over previous
import functools
import jax
import jax.numpy as jnp
from jax import lax
from jax.experimental import pallas as pl
from jax.experimental.pallas import tpu as pltpu

N = 10000
E = 320000
D_IN = 128
D_EDGE = 16
H = 128
L = 3
G = 64
NT = 12
TE = 32
BN_EPS = 1e-5

_NB = 5
_BLK = N // _NB


def _head_body(batch_ref, tidx_ref, h_ref, Temb_ref, Wh1_ref, bh1_ref,
               Wh2_ref, bh2_ref, Wh3_ref, bh3_ref, out_ref, acc_ref):
    i = pl.program_id(0)
    nb = pl.num_programs(0)

    @pl.when(i == 0)
    def _init():
        acc_ref[...] = jnp.zeros_like(acc_ref)

    b = batch_ref[0, 0]
    h = h_ref[...]
    gids = lax.broadcasted_iota(jnp.int32, (G, _BLK), 0)
    onehot = (gids == b[None, :]).astype(jnp.float32)
    acc_ref[...] += jnp.dot(onehot, h, preferred_element_type=jnp.float32)

    @pl.when(i == nb - 1)
    def _fin():
        g = acc_ref[...]
        t = tidx_ref[0]
        te_oh = (lax.broadcasted_iota(jnp.int32, (G, NT), 1)
                 == t[:, None]).astype(jnp.float32)
        temb = jnp.dot(te_oh, Temb_ref[...],
                       preferred_element_type=jnp.float32)
        r1 = (jnp.dot(g, Wh1_ref[:H, :], preferred_element_type=jnp.float32)
              + jnp.dot(temb, Wh1_ref[H:, :],
                        preferred_element_type=jnp.float32)
              + bh1_ref[...])
        r1 = jnp.maximum(r1, 0.0)
        r2 = jnp.dot(r1, Wh2_ref[...], preferred_element_type=jnp.float32)
        r2 = jnp.maximum(r2 + bh2_ref[...], 0.0)
        r3 = jnp.dot(r2, Wh3_ref[...], preferred_element_type=jnp.float32)
        out_ref[...] = (r3 + bh3_ref[...]).reshape(1, G)


def _head(h, batch, target_idx, Temb, Wh1, bh1, Wh2, bh2, Wh3, bh3):
    batch3 = batch.reshape(_NB, 1, _BLK)
    tidx2 = target_idx.reshape(1, G)
    out = pl.pallas_call(
        _head_body,
        grid=(_NB,),
        in_specs=[
            pl.BlockSpec((1, 1, _BLK), lambda i: (i, 0, 0)),
            pl.BlockSpec((1, G), lambda i: (0, 0)),
            pl.BlockSpec((_BLK, H), lambda i: (i, 0)),
            pl.BlockSpec((NT, TE), lambda i: (0, 0)),
            pl.BlockSpec((H + TE, H), lambda i: (0, 0)),
            pl.BlockSpec((1, H), lambda i: (0, 0)),
            pl.BlockSpec((H, H // 2), lambda i: (0, 0)),
            pl.BlockSpec((1, H // 2), lambda i: (0, 0)),
            pl.BlockSpec((H // 2, 1), lambda i: (0, 0)),
            pl.BlockSpec((1, 1), lambda i: (0, 0)),
        ],
        out_specs=pl.BlockSpec((1, G), lambda i: (0, 0)),
        out_shape=jax.ShapeDtypeStruct((1, G), jnp.float32),
        scratch_shapes=[pltpu.VMEM((G, H), jnp.float32)],
    )(batch3, tidx2, h, Temb, Wh1, bh1.reshape(1, H), Wh2,
      bh2.reshape(1, H // 2), Wh3, bh3.reshape(1, 1))
    return out.reshape(G)


def kernel(x, edge_index, edge_attr, batch, target_idx, Wp, bp, eps, We, be,
           W1, b1, W2, b2, gamma, beta, Temb, Wh1, bh1, Wh2, bh2, Wh3, bh3):
    src = edge_index[0]
    dst = edge_index[1]
    h = x @ Wp + bp
    for l in range(L):
        ea = edge_attr @ We[l] + be[l]
        m = jax.nn.relu(h[src] + ea)
        aggr = jax.ops.segment_sum(m, dst, num_segments=N)
        z = (1.0 + eps[l]) * h + aggr
        z = jax.nn.relu(z @ W1[l] + b1[l]) @ W2[l] + b2[l]
        z = z / jnp.sqrt(1.0 + BN_EPS) * gamma[l] + beta[l]
        h = jax.nn.relu(z)
    return _head(h, batch, target_idx, Temb, Wh1, bh1, Wh2, bh2, Wh3, bh3)


# speedup vs baseline: 2.4108x; 2.3777x over previous
import functools
import jax
import jax.numpy as jnp
from jax import lax
from jax.experimental import pallas as pl
from jax.experimental.pallas import tpu as pltpu
from jax.experimental.pallas import tpu_sc as plsc

N = 10000
E = 320000
D_IN = 128
D_EDGE = 16
H = 128
L = 3
G = 64
NT = 12
TE = 32
BN_EPS = 1e-5

_NB = 5
_BLK = N // _NB

_NW = 32
_EPW = E // _NW
_EC = 80
_NCH = _EPW // _EC
_RPT = 624
_ZR = 208


def _h0_body(x_ref, Wp_ref, bp_ref, out_ref):
    out_ref[...] = jnp.dot(x_ref[...], Wp_ref[...],
                           preferred_element_type=jnp.float32) + bp_ref[...]


def _h0(x, Wp, bp):
    return pl.pallas_call(
        _h0_body,
        grid=(_NB,),
        in_specs=[
            pl.BlockSpec((_BLK, D_IN), lambda i: (i, 0)),
            pl.BlockSpec((D_IN, H), lambda i: (0, 0)),
            pl.BlockSpec((1, H), lambda i: (0, 0)),
        ],
        out_specs=pl.BlockSpec((_BLK, H), lambda i: (i, 0)),
        out_shape=jax.ShapeDtypeStruct((N, H), jnp.float32),
    )(x, Wp, bp.reshape(1, H))


_BE = 8000


def _ea_body(eattr_ref, We_ref, be_ref, out_ref):
    out_ref[0] = jnp.dot(eattr_ref[...], We_ref[0],
                         preferred_element_type=jnp.float32) + be_ref[0, 0]


def _ea_all(edge_attr, We, be):
    return pl.pallas_call(
        _ea_body,
        grid=(L, E // _BE),
        in_specs=[
            pl.BlockSpec((_BE, D_EDGE), lambda l, e: (e, 0)),
            pl.BlockSpec((1, D_EDGE, H), lambda l, e: (l, 0, 0)),
            pl.BlockSpec((1, 1, H), lambda l, e: (l, 0, 0)),
        ],
        out_specs=pl.BlockSpec((1, _BE, H), lambda l, e: (l, e, 0)),
        out_shape=jax.ShapeDtypeStruct((L, E, H), jnp.float32),
    )(edge_attr, We, be.reshape(L, 1, H))


def _edge_layer(h, ea_all, src, dst, l):
    mesh = plsc.VectorSubcoreMesh(core_axis_name="c", subcore_axis_name="s")

    @functools.partial(
        pl.kernel,
        mesh=mesh,
        out_type=jax.ShapeDtypeStruct((2, N, H), jnp.float32),
        scratch_types=[
            pltpu.VMEM((_EC,), jnp.int32),
            pltpu.VMEM((_EC,), jnp.int32),
            pltpu.VMEM((_EC, H), jnp.float32),
            pltpu.VMEM((_EC, H), jnp.float32),
            pltpu.VMEM((_ZR, H), jnp.float32),
            pltpu.VMEM_SHARED((N, H), jnp.float32),
            pltpu.SemaphoreType.DMA,
        ],
    )
    def body(h_hbm, ea_hbm, src_hbm, dst_hbm, out_hbm,
             srcv, dstv, mv, eav, zv, aggr, sem):
        c = lax.axis_index("c")
        s = lax.axis_index("s")
        wid = s * 2 + c
        zero16 = jnp.zeros((16,), jnp.float32)

        def zrow(r, _):
            for j in range(H // 16):
                zv[r, pl.ds(j * 16, 16)] = zero16
            return 0

        lax.fori_loop(0, _ZR, zrow, 0)

        def zcopy(k, _):
            pltpu.sync_copy(zv, aggr.at[pl.ds(s * _RPT + k * _ZR, _ZR)])
            return 0

        lax.fori_loop(0, _RPT // _ZR, zcopy, 0)

        @pl.when(s == 15)
        def _ztail():
            pltpu.sync_copy(zv.at[pl.ds(0, 16)], aggr.at[pl.ds(16 * _RPT, 16)])

        plsc.subcore_barrier()

        def chunk(ci, _):
            base = wid * _EPW + ci * _EC
            pltpu.sync_copy(src_hbm.at[pl.ds(base, _EC)], srcv)
            pltpu.sync_copy(dst_hbm.at[pl.ds(base, _EC)], dstv)
            pltpu.async_copy(h_hbm.at[srcv], mv, sem).wait()
            pltpu.sync_copy(ea_hbm.at[l, pl.ds(base, _EC)], eav)

            def mrow(e, _):
                for j in range(H // 16):
                    sl = pl.ds(j * 16, 16)
                    mv[e, sl] = jnp.maximum(mv[e, sl] + eav[e, sl], 0.0)
                return 0

            lax.fori_loop(0, _EC, mrow, 0)
            pltpu.sync_copy(mv, aggr.at[dstv], add=True)
            return 0

        lax.fori_loop(0, _NCH, chunk, 0)
        plsc.subcore_barrier()
        pltpu.sync_copy(aggr.at[pl.ds(s * _RPT, _RPT)],
                        out_hbm.at[c, pl.ds(s * _RPT, _RPT)])

        @pl.when(s == 15)
        def _wtail():
            pltpu.sync_copy(aggr.at[pl.ds(16 * _RPT, 16)],
                            out_hbm.at[c, pl.ds(16 * _RPT, 16)])

    return body(h, ea_all, src, dst)


def _node_body(eps_ref, h_ref, a_ref, W1_ref, b1_ref, W2_ref, b2_ref,
               gam_ref, bet_ref, out_ref):
    z = (1.0 + eps_ref[0, 0]) * h_ref[...] + a_ref[0] + a_ref[1]
    t = jnp.maximum(
        jnp.dot(z, W1_ref[0], preferred_element_type=jnp.float32)
        + b1_ref[0, 0], 0.0)
    z2 = (jnp.dot(t, W2_ref[0], preferred_element_type=jnp.float32)
          + b2_ref[0, 0])
    out_ref[...] = jnp.maximum(z2 * gam_ref[0, 0] + bet_ref[0, 0], 0.0)


def _node(l, eps1, h, aggr, W1, b1, W2, b2, gam_s, beta):
    return pl.pallas_call(
        functools.partial(_node_body),
        grid=(_NB,),
        in_specs=[
            pl.BlockSpec((1, 1), lambda i: (0, 0)),
            pl.BlockSpec((_BLK, H), lambda i: (i, 0)),
            pl.BlockSpec((2, _BLK, H), lambda i: (0, i, 0)),
            pl.BlockSpec((1, H, H), lambda i: (l, 0, 0)),
            pl.BlockSpec((1, 1, H), lambda i: (l, 0, 0)),
            pl.BlockSpec((1, H, H), lambda i: (l, 0, 0)),
            pl.BlockSpec((1, 1, H), lambda i: (l, 0, 0)),
            pl.BlockSpec((1, 1, H), lambda i: (l, 0, 0)),
            pl.BlockSpec((1, 1, H), lambda i: (l, 0, 0)),
        ],
        out_specs=pl.BlockSpec((_BLK, H), lambda i: (i, 0)),
        out_shape=jax.ShapeDtypeStruct((N, H), jnp.float32),
    )(eps1, h, aggr, W1, b1.reshape(L, 1, H), W2, b2.reshape(L, 1, H),
      gam_s.reshape(L, 1, H), beta.reshape(L, 1, H))


def _head_body(batch_ref, tidx_ref, h_ref, Temb_ref, Wh1_ref, bh1_ref,
               Wh2_ref, bh2_ref, Wh3_ref, bh3_ref, out_ref, acc_ref):
    i = pl.program_id(0)
    nb = pl.num_programs(0)

    @pl.when(i == 0)
    def _init():
        acc_ref[...] = jnp.zeros_like(acc_ref)

    b = batch_ref[0, 0]
    h = h_ref[...]
    gids = lax.broadcasted_iota(jnp.int32, (G, _BLK), 0)
    onehot = (gids == b[None, :]).astype(jnp.float32)
    acc_ref[...] += jnp.dot(onehot, h, preferred_element_type=jnp.float32)

    @pl.when(i == nb - 1)
    def _fin():
        g = acc_ref[...]
        t = tidx_ref[0]
        te_oh = (lax.broadcasted_iota(jnp.int32, (G, NT), 1)
                 == t[:, None]).astype(jnp.float32)
        temb = jnp.dot(te_oh, Temb_ref[...],
                       preferred_element_type=jnp.float32)
        r1 = (jnp.dot(g, Wh1_ref[:H, :], preferred_element_type=jnp.float32)
              + jnp.dot(temb, Wh1_ref[H:, :],
                        preferred_element_type=jnp.float32)
              + bh1_ref[...])
        r1 = jnp.maximum(r1, 0.0)
        r2 = jnp.dot(r1, Wh2_ref[...], preferred_element_type=jnp.float32)
        r2 = jnp.maximum(r2 + bh2_ref[...], 0.0)
        r3 = jnp.dot(r2, Wh3_ref[...], preferred_element_type=jnp.float32)
        out_ref[...] = (r3 + bh3_ref[...]).reshape(1, G)


def _head(h, batch, target_idx, Temb, Wh1, bh1, Wh2, bh2, Wh3, bh3):
    batch3 = batch.reshape(_NB, 1, _BLK)
    tidx2 = target_idx.reshape(1, G)
    out = pl.pallas_call(
        _head_body,
        grid=(_NB,),
        in_specs=[
            pl.BlockSpec((1, 1, _BLK), lambda i: (i, 0, 0)),
            pl.BlockSpec((1, G), lambda i: (0, 0)),
            pl.BlockSpec((_BLK, H), lambda i: (i, 0)),
            pl.BlockSpec((NT, TE), lambda i: (0, 0)),
            pl.BlockSpec((H + TE, H), lambda i: (0, 0)),
            pl.BlockSpec((1, H), lambda i: (0, 0)),
            pl.BlockSpec((H, H // 2), lambda i: (0, 0)),
            pl.BlockSpec((1, H // 2), lambda i: (0, 0)),
            pl.BlockSpec((H // 2, 1), lambda i: (0, 0)),
            pl.BlockSpec((1, 1), lambda i: (0, 0)),
        ],
        out_specs=pl.BlockSpec((1, G), lambda i: (0, 0)),
        out_shape=jax.ShapeDtypeStruct((1, G), jnp.float32),
        scratch_shapes=[pltpu.VMEM((G, H), jnp.float32)],
    )(batch3, tidx2, h, Temb, Wh1, bh1.reshape(1, H), Wh2,
      bh2.reshape(1, H // 2), Wh3, bh3.reshape(1, 1))
    return out.reshape(G)


def kernel(x, edge_index, edge_attr, batch, target_idx, Wp, bp, eps, We, be,
           W1, b1, W2, b2, gamma, beta, Temb, Wh1, bh1, Wh2, bh2, Wh3, bh3):
    src = edge_index[0]
    dst = edge_index[1]
    h = _h0(x, Wp, bp)
    ea_all = _ea_all(edge_attr, We, be)
    gam_s = gamma / jnp.sqrt(1.0 + BN_EPS)
    for l in range(L):
        aggr = _edge_layer(h, ea_all, src, dst, l)
        eps1 = eps[l].reshape(1, 1)
        h = _node(l, eps1, h, aggr, W1, b1, W2, b2, gam_s, beta)
    return _head(h, batch, target_idx, Temb, Wh1, bh1, Wh2, bh2, Wh3, bh3)


# speedup vs baseline: 3.7701x; 1.5639x over previous
import functools
import jax
import jax.numpy as jnp
from jax import lax
from jax.experimental import pallas as pl
from jax.experimental.pallas import tpu as pltpu
from jax.experimental.pallas import tpu_sc as plsc

N = 10000
E = 320000
D_IN = 128
D_EDGE = 16
H = 128
L = 3
G = 64
NT = 12
TE = 32
BN_EPS = 1e-5

_NB = 5
_BLK = N // _NB

_NW = 32
_EPW = E // _NW
_EC = 80
_NCH = _EPW // _EC
_RPT = 624
_ZR = 16


def _h0_body(x_ref, Wp_ref, bp_ref, out_ref):
    out_ref[...] = jnp.dot(x_ref[...], Wp_ref[...],
                           preferred_element_type=jnp.float32) + bp_ref[...]


def _h0(x, Wp, bp):
    return pl.pallas_call(
        _h0_body,
        grid=(_NB,),
        in_specs=[
            pl.BlockSpec((_BLK, D_IN), lambda i: (i, 0)),
            pl.BlockSpec((D_IN, H), lambda i: (0, 0)),
            pl.BlockSpec((1, H), lambda i: (0, 0)),
        ],
        out_specs=pl.BlockSpec((_BLK, H), lambda i: (i, 0)),
        out_shape=jax.ShapeDtypeStruct((N, H), jnp.float32),
    )(x, Wp, bp.reshape(1, H))


_BE = 8000


def _ea_body(eattr_ref, We_ref, be_ref, out_ref):
    out_ref[0] = jnp.dot(eattr_ref[...], We_ref[0],
                         preferred_element_type=jnp.float32) + be_ref[0, 0]


def _ea_all(edge_attr, We, be):
    return pl.pallas_call(
        _ea_body,
        grid=(L, E // _BE),
        in_specs=[
            pl.BlockSpec((_BE, D_EDGE), lambda l, e: (e, 0)),
            pl.BlockSpec((1, D_EDGE, H), lambda l, e: (l, 0, 0)),
            pl.BlockSpec((1, 1, H), lambda l, e: (l, 0, 0)),
        ],
        out_specs=pl.BlockSpec((1, _BE, H), lambda l, e: (l, e, 0)),
        out_shape=jax.ShapeDtypeStruct((L, E, H), jnp.float32),
    )(edge_attr, We, be.reshape(L, 1, H))


def _edge_layer(h, ea_all, src, dst, l):
    mesh = plsc.VectorSubcoreMesh(core_axis_name="c", subcore_axis_name="s")

    @functools.partial(
        pl.kernel,
        mesh=mesh,
        out_type=jax.ShapeDtypeStruct((2, N, H), jnp.float32),
        scratch_types=[
            pltpu.VMEM((_EC,), jnp.int32),
            pltpu.VMEM((_EC,), jnp.int32),
            pltpu.VMEM((_EC,), jnp.int32),
            pltpu.VMEM((_EC,), jnp.int32),
            pltpu.VMEM((_EC, H), jnp.float32),
            pltpu.VMEM((_EC, H), jnp.float32),
            pltpu.VMEM((_EC, H), jnp.float32),
            pltpu.VMEM((_EC, H), jnp.float32),
            pltpu.VMEM((_EC,), jnp.int32),
            pltpu.VMEM((_EC,), jnp.int32),
            pltpu.VMEM((_ZR, H), jnp.float32),
            pltpu.VMEM_SHARED((N, H), jnp.float32),
            pltpu.SemaphoreType.DMA,
            pltpu.SemaphoreType.DMA,
            pltpu.SemaphoreType.DMA,
            pltpu.SemaphoreType.DMA,
            pltpu.SemaphoreType.DMA,
            pltpu.SemaphoreType.DMA,
            pltpu.SemaphoreType.DMA,
            pltpu.SemaphoreType.DMA,
        ],
    )
    def body(h_hbm, ea_hbm, src_hbm, dst_hbm, out_hbm,
             srcv0, srcv1, dstv0, dstv1, mv0, mv1, eav0, eav1,
             dsts0, dsts1, zv, aggr,
             isem0, isem1, esem0, esem1, gsem0, gsem1, ssem0, ssem1):
        c = lax.axis_index("c")
        s = lax.axis_index("s")
        wid = s * 2 + c
        srcv = (srcv0, srcv1)
        dstv = (dstv0, dstv1)
        dsts = (dsts0, dsts1)
        mv = (mv0, mv1)
        eav = (eav0, eav1)
        isem = (isem0, isem1)
        esem = (esem0, esem1)
        gsem = (gsem0, gsem1)
        ssem = (ssem0, ssem1)

        def start_loads(ci, b):
            base = wid * _EPW + ci * _EC
            pltpu.async_copy(src_hbm.at[pl.ds(base, _EC)], srcv[b], isem[b])
            pltpu.async_copy(dst_hbm.at[pl.ds(base, _EC)], dstv[b], isem[b])
            pltpu.async_copy(ea_hbm.at[l, pl.ds(base, _EC)], eav[b], esem[b])

        def wait_idx(b):
            pltpu.make_async_copy(src_hbm.at[pl.ds(0, _EC)], srcv[b],
                                  isem[b]).wait()
            pltpu.make_async_copy(dst_hbm.at[pl.ds(0, _EC)], dstv[b],
                                  isem[b]).wait()

        def start_gather(b):
            pltpu.async_copy(h_hbm.at[srcv[b]], mv[b], gsem[b])

        def wait_gather_ea(b):
            pltpu.make_async_copy(h_hbm.at[srcv[b]], mv[b], gsem[b]).wait()
            pltpu.make_async_copy(ea_hbm.at[l, pl.ds(0, _EC)], eav[b],
                                  esem[b]).wait()

        def start_scatter(b):
            for k in range(_EC // 16):
                sl = pl.ds(k * 16, 16)
                dsts[b][sl] = dstv[b][sl]
            pltpu.async_copy(mv[b], aggr.at[dsts[b]], ssem[b], add=True)

        def wait_scatter(b):
            pltpu.make_async_copy(mv[b], aggr.at[dsts[b]], ssem[b]).wait()

        def compute(b):
            def mrow(e, _):
                for j in range(H // 16):
                    sl = pl.ds(j * 16, 16)
                    mv[b][e, sl] = jnp.maximum(mv[b][e, sl] + eav[b][e, sl],
                                               0.0)
                return 0

            lax.fori_loop(0, _EC, mrow, 0)

        start_loads(0, 0)
        start_loads(1, 1)
        zero16 = jnp.zeros((16,), jnp.float32)

        def zrow(r, _):
            for j in range(H // 16):
                zv[r, pl.ds(j * 16, 16)] = zero16
            return 0

        lax.fori_loop(0, _ZR, zrow, 0)

        def zcopy(k, _):
            pltpu.sync_copy(zv, aggr.at[pl.ds(s * _RPT + k * _ZR, _ZR)])
            return 0

        lax.fori_loop(0, _RPT // _ZR, zcopy, 0)

        @pl.when(s == 15)
        def _ztail():
            pltpu.sync_copy(zv, aggr.at[pl.ds(16 * _RPT, 16)])

        plsc.subcore_barrier()
        wait_idx(0)
        start_gather(0)

        def step(jj, _):
            for b in range(2):
                ci = 2 * jj + b
                wait_gather_ea(b)
                compute(b)
                start_scatter(b)

                @pl.when(ci + 2 < _NCH)
                def _ld():
                    start_loads(ci + 2, b)

                @pl.when(ci >= 1)
                def _ws():
                    wait_scatter(1 - b)

                @pl.when(ci + 1 < _NCH)
                def _g():
                    wait_idx(1 - b)
                    start_gather(1 - b)
            return 0

        lax.fori_loop(0, (_NCH - 1) // 2, step, 0)

        wait_gather_ea(0)
        compute(0)
        start_scatter(0)
        wait_scatter(1)
        wait_scatter(0)
        plsc.subcore_barrier()
        pltpu.sync_copy(aggr.at[pl.ds(s * _RPT, _RPT)],
                        out_hbm.at[c, pl.ds(s * _RPT, _RPT)])

        @pl.when(s == 15)
        def _wtail():
            pltpu.sync_copy(aggr.at[pl.ds(16 * _RPT, 16)],
                            out_hbm.at[c, pl.ds(16 * _RPT, 16)])

    return body(h, ea_all, src, dst)


def _node_body(eps_ref, h_ref, a_ref, W1_ref, b1_ref, W2_ref, b2_ref,
               gam_ref, bet_ref, out_ref):
    z = (1.0 + eps_ref[0, 0]) * h_ref[...] + a_ref[0] + a_ref[1]
    t = jnp.maximum(
        jnp.dot(z, W1_ref[0], preferred_element_type=jnp.float32)
        + b1_ref[0, 0], 0.0)
    z2 = (jnp.dot(t, W2_ref[0], preferred_element_type=jnp.float32)
          + b2_ref[0, 0])
    out_ref[...] = jnp.maximum(z2 * gam_ref[0, 0] + bet_ref[0, 0], 0.0)


def _node(l, eps1, h, aggr, W1, b1, W2, b2, gam_s, beta):
    return pl.pallas_call(
        functools.partial(_node_body),
        grid=(_NB,),
        in_specs=[
            pl.BlockSpec((1, 1), lambda i: (0, 0)),
            pl.BlockSpec((_BLK, H), lambda i: (i, 0)),
            pl.BlockSpec((2, _BLK, H), lambda i: (0, i, 0)),
            pl.BlockSpec((1, H, H), lambda i: (l, 0, 0)),
            pl.BlockSpec((1, 1, H), lambda i: (l, 0, 0)),
            pl.BlockSpec((1, H, H), lambda i: (l, 0, 0)),
            pl.BlockSpec((1, 1, H), lambda i: (l, 0, 0)),
            pl.BlockSpec((1, 1, H), lambda i: (l, 0, 0)),
            pl.BlockSpec((1, 1, H), lambda i: (l, 0, 0)),
        ],
        out_specs=pl.BlockSpec((_BLK, H), lambda i: (i, 0)),
        out_shape=jax.ShapeDtypeStruct((N, H), jnp.float32),
    )(eps1, h, aggr, W1, b1.reshape(L, 1, H), W2, b2.reshape(L, 1, H),
      gam_s.reshape(L, 1, H), beta.reshape(L, 1, H))


def _head_body(batch_ref, tidx_ref, h_ref, Temb_ref, Wh1_ref, bh1_ref,
               Wh2_ref, bh2_ref, Wh3_ref, bh3_ref, out_ref, acc_ref):
    i = pl.program_id(0)
    nb = pl.num_programs(0)

    @pl.when(i == 0)
    def _init():
        acc_ref[...] = jnp.zeros_like(acc_ref)

    b = batch_ref[0, 0]
    h = h_ref[...]
    gids = lax.broadcasted_iota(jnp.int32, (G, _BLK), 0)
    onehot = (gids == b[None, :]).astype(jnp.float32)
    acc_ref[...] += jnp.dot(onehot, h, preferred_element_type=jnp.float32)

    @pl.when(i == nb - 1)
    def _fin():
        g = acc_ref[...]
        t = tidx_ref[0]
        te_oh = (lax.broadcasted_iota(jnp.int32, (G, NT), 1)
                 == t[:, None]).astype(jnp.float32)
        temb = jnp.dot(te_oh, Temb_ref[...],
                       preferred_element_type=jnp.float32)
        r1 = (jnp.dot(g, Wh1_ref[:H, :], preferred_element_type=jnp.float32)
              + jnp.dot(temb, Wh1_ref[H:, :],
                        preferred_element_type=jnp.float32)
              + bh1_ref[...])
        r1 = jnp.maximum(r1, 0.0)
        r2 = jnp.dot(r1, Wh2_ref[...], preferred_element_type=jnp.float32)
        r2 = jnp.maximum(r2 + bh2_ref[...], 0.0)
        r3 = jnp.dot(r2, Wh3_ref[...], preferred_element_type=jnp.float32)
        out_ref[...] = (r3 + bh3_ref[...]).reshape(1, G)


def _head(h, batch, target_idx, Temb, Wh1, bh1, Wh2, bh2, Wh3, bh3):
    batch3 = batch.reshape(_NB, 1, _BLK)
    tidx2 = target_idx.reshape(1, G)
    out = pl.pallas_call(
        _head_body,
        grid=(_NB,),
        in_specs=[
            pl.BlockSpec((1, 1, _BLK), lambda i: (i, 0, 0)),
            pl.BlockSpec((1, G), lambda i: (0, 0)),
            pl.BlockSpec((_BLK, H), lambda i: (i, 0)),
            pl.BlockSpec((NT, TE), lambda i: (0, 0)),
            pl.BlockSpec((H + TE, H), lambda i: (0, 0)),
            pl.BlockSpec((1, H), lambda i: (0, 0)),
            pl.BlockSpec((H, H // 2), lambda i: (0, 0)),
            pl.BlockSpec((1, H // 2), lambda i: (0, 0)),
            pl.BlockSpec((H // 2, 1), lambda i: (0, 0)),
            pl.BlockSpec((1, 1), lambda i: (0, 0)),
        ],
        out_specs=pl.BlockSpec((1, G), lambda i: (0, 0)),
        out_shape=jax.ShapeDtypeStruct((1, G), jnp.float32),
        scratch_shapes=[pltpu.VMEM((G, H), jnp.float32)],
    )(batch3, tidx2, h, Temb, Wh1, bh1.reshape(1, H), Wh2,
      bh2.reshape(1, H // 2), Wh3, bh3.reshape(1, 1))
    return out.reshape(G)


def kernel(x, edge_index, edge_attr, batch, target_idx, Wp, bp, eps, We, be,
           W1, b1, W2, b2, gamma, beta, Temb, Wh1, bh1, Wh2, bh2, Wh3, bh3):
    src = edge_index[0]
    dst = edge_index[1]
    h = _h0(x, Wp, bp)
    ea_all = _ea_all(edge_attr, We, be)
    gam_s = gamma / jnp.sqrt(1.0 + BN_EPS)
    for l in range(L):
        aggr = _edge_layer(h, ea_all, src, dst, l)
        eps1 = eps[l].reshape(1, 1)
        h = _node(l, eps1, h, aggr, W1, b1, W2, b2, gam_s, beta)
    return _head(h, batch, target_idx, Temb, Wh1, bh1, Wh2, bh2, Wh3, bh3)


# speedup vs baseline: 4.1347x; 1.0967x over previous
import functools
import jax
import jax.numpy as jnp
from jax import lax
from jax.experimental import pallas as pl
from jax.experimental.pallas import tpu as pltpu
from jax.experimental.pallas import tpu_sc as plsc

N = 10000
E = 320000
D_IN = 128
D_EDGE = 16
H = 128
L = 3
G = 64
NT = 12
TE = 32
BN_EPS = 1e-5

_NB = 5
_BLK = N // _NB

_NW = 32
_EPW = E // _NW
_EC = 80
_NCH = _EPW // _EC
_RPT = 624
_ZR = 16


def _h0_body(x_ref, Wp_ref, bp_ref, out_ref):
    out_ref[...] = jnp.dot(x_ref[...], Wp_ref[...],
                           preferred_element_type=jnp.float32) + bp_ref[...]


def _h0(x, Wp, bp):
    return pl.pallas_call(
        _h0_body,
        grid=(_NB,),
        in_specs=[
            pl.BlockSpec((_BLK, D_IN), lambda i: (i, 0)),
            pl.BlockSpec((D_IN, H), lambda i: (0, 0)),
            pl.BlockSpec((1, H), lambda i: (0, 0)),
        ],
        out_specs=pl.BlockSpec((_BLK, H), lambda i: (i, 0)),
        out_shape=jax.ShapeDtypeStruct((N, H), jnp.float32),
    )(x, Wp, bp.reshape(1, H))


_BE = 8000


def _ea_body(eattr_ref, We_ref, be_ref, out_ref):
    out_ref[...] = jnp.dot(eattr_ref[...], We_ref[0],
                           preferred_element_type=jnp.float32) + be_ref[0, 0]


def _ea_layer(edge_attr, We, be, l):
    return pl.pallas_call(
        _ea_body,
        grid=(E // _BE,),
        in_specs=[
            pl.BlockSpec((_BE, D_EDGE), lambda e: (e, 0)),
            pl.BlockSpec((1, D_EDGE, H), lambda e: (l, 0, 0)),
            pl.BlockSpec((1, 1, H), lambda e: (l, 0, 0)),
        ],
        out_specs=pl.BlockSpec((_BE, H), lambda e: (e, 0)),
        out_shape=jax.ShapeDtypeStruct((E, H), jnp.float32),
    )(edge_attr, We, be.reshape(L, 1, H))


def _edge_layer(h, ea_l, src, dst):
    mesh = plsc.VectorSubcoreMesh(core_axis_name="c", subcore_axis_name="s")

    @functools.partial(
        pl.kernel,
        mesh=mesh,
        out_type=jax.ShapeDtypeStruct((2, N, H), jnp.float32),
        scratch_types=[
            pltpu.VMEM((_EC,), jnp.int32),
            pltpu.VMEM((_EC,), jnp.int32),
            pltpu.VMEM((_EC,), jnp.int32),
            pltpu.VMEM((_EC,), jnp.int32),
            pltpu.VMEM((_EC, H), jnp.float32),
            pltpu.VMEM((_EC, H), jnp.float32),
            pltpu.VMEM((_EC, H), jnp.float32),
            pltpu.VMEM((_EC, H), jnp.float32),
            pltpu.VMEM((_EC,), jnp.int32),
            pltpu.VMEM((_EC,), jnp.int32),
            pltpu.VMEM((_ZR, H), jnp.float32),
            pltpu.VMEM_SHARED((N, H), jnp.float32),
            pltpu.SemaphoreType.DMA,
            pltpu.SemaphoreType.DMA,
            pltpu.SemaphoreType.DMA,
            pltpu.SemaphoreType.DMA,
            pltpu.SemaphoreType.DMA,
            pltpu.SemaphoreType.DMA,
            pltpu.SemaphoreType.DMA,
            pltpu.SemaphoreType.DMA,
        ],
    )
    def body(h_hbm, ea_hbm, src_hbm, dst_hbm, out_hbm,
             srcv0, srcv1, dstv0, dstv1, mv0, mv1, eav0, eav1,
             dsts0, dsts1, zv, aggr,
             isem0, isem1, esem0, esem1, gsem0, gsem1, ssem0, ssem1):
        c = lax.axis_index("c")
        s = lax.axis_index("s")
        wid = s * 2 + c
        srcv = (srcv0, srcv1)
        dstv = (dstv0, dstv1)
        dsts = (dsts0, dsts1)
        mv = (mv0, mv1)
        eav = (eav0, eav1)
        isem = (isem0, isem1)
        esem = (esem0, esem1)
        gsem = (gsem0, gsem1)
        ssem = (ssem0, ssem1)

        def start_loads(ci, b):
            base = wid * _EPW + ci * _EC
            pltpu.async_copy(src_hbm.at[pl.ds(base, _EC)], srcv[b], isem[b])
            pltpu.async_copy(dst_hbm.at[pl.ds(base, _EC)], dstv[b], isem[b])
            pltpu.async_copy(ea_hbm.at[pl.ds(base, _EC)], eav[b], esem[b])

        def wait_idx(b):
            pltpu.make_async_copy(src_hbm.at[pl.ds(0, _EC)], srcv[b],
                                  isem[b]).wait()
            pltpu.make_async_copy(dst_hbm.at[pl.ds(0, _EC)], dstv[b],
                                  isem[b]).wait()

        def start_gather(b):
            pltpu.async_copy(h_hbm.at[srcv[b]], mv[b], gsem[b])

        def wait_gather_ea(b):
            pltpu.make_async_copy(h_hbm.at[srcv[b]], mv[b], gsem[b]).wait()
            pltpu.make_async_copy(ea_hbm.at[pl.ds(0, _EC)], eav[b],
                                  esem[b]).wait()

        def start_scatter(b):
            for k in range(_EC // 16):
                sl = pl.ds(k * 16, 16)
                dsts[b][sl] = dstv[b][sl]
            pltpu.async_copy(mv[b], aggr.at[dsts[b]], ssem[b], add=True)

        def wait_scatter(b):
            pltpu.make_async_copy(mv[b], aggr.at[dsts[b]], ssem[b]).wait()

        def compute(b):
            def mrow(e, _):
                for j in range(H // 16):
                    sl = pl.ds(j * 16, 16)
                    mv[b][e, sl] = jnp.maximum(mv[b][e, sl] + eav[b][e, sl],
                                               0.0)
                return 0

            lax.fori_loop(0, _EC, mrow, 0)

        start_loads(0, 0)
        start_loads(1, 1)
        zero16 = jnp.zeros((16,), jnp.float32)

        def zrow(r, _):
            for j in range(H // 16):
                zv[r, pl.ds(j * 16, 16)] = zero16
            return 0

        lax.fori_loop(0, _ZR, zrow, 0)

        def zcopy(k, _):
            pltpu.sync_copy(zv, aggr.at[pl.ds(s * _RPT + k * _ZR, _ZR)])
            return 0

        lax.fori_loop(0, _RPT // _ZR, zcopy, 0)

        @pl.when(s == 15)
        def _ztail():
            pltpu.sync_copy(zv, aggr.at[pl.ds(16 * _RPT, 16)])

        plsc.subcore_barrier()
        wait_idx(0)
        start_gather(0)

        def step(jj, _):
            for b in range(2):
                ci = 2 * jj + b
                wait_gather_ea(b)
                compute(b)
                start_scatter(b)

                @pl.when(ci + 2 < _NCH)
                def _ld():
                    start_loads(ci + 2, b)

                @pl.when(ci >= 1)
                def _ws():
                    wait_scatter(1 - b)

                @pl.when(ci + 1 < _NCH)
                def _g():
                    wait_idx(1 - b)
                    start_gather(1 - b)
            return 0

        lax.fori_loop(0, (_NCH - 1) // 2, step, 0)

        wait_gather_ea(0)
        compute(0)
        start_scatter(0)
        wait_scatter(1)
        wait_scatter(0)
        plsc.subcore_barrier()
        pltpu.sync_copy(aggr.at[pl.ds(s * _RPT, _RPT)],
                        out_hbm.at[c, pl.ds(s * _RPT, _RPT)])

        @pl.when(s == 15)
        def _wtail():
            pltpu.sync_copy(aggr.at[pl.ds(16 * _RPT, 16)],
                            out_hbm.at[c, pl.ds(16 * _RPT, 16)])

    return body(h, ea_l, src, dst)


def _node_body(eps_ref, h_ref, a_ref, W1_ref, b1_ref, W2_ref, b2_ref,
               gam_ref, bet_ref, out_ref):
    z = (1.0 + eps_ref[0, 0]) * h_ref[...] + a_ref[0] + a_ref[1]
    t = jnp.maximum(
        jnp.dot(z, W1_ref[0], preferred_element_type=jnp.float32)
        + b1_ref[0, 0], 0.0)
    z2 = (jnp.dot(t, W2_ref[0], preferred_element_type=jnp.float32)
          + b2_ref[0, 0])
    out_ref[...] = jnp.maximum(z2 * gam_ref[0, 0] + bet_ref[0, 0], 0.0)


def _node(l, eps1, h, aggr, W1, b1, W2, b2, gam_s, beta):
    return pl.pallas_call(
        functools.partial(_node_body),
        grid=(_NB,),
        in_specs=[
            pl.BlockSpec((1, 1), lambda i: (0, 0)),
            pl.BlockSpec((_BLK, H), lambda i: (i, 0)),
            pl.BlockSpec((2, _BLK, H), lambda i: (0, i, 0)),
            pl.BlockSpec((1, H, H), lambda i: (l, 0, 0)),
            pl.BlockSpec((1, 1, H), lambda i: (l, 0, 0)),
            pl.BlockSpec((1, H, H), lambda i: (l, 0, 0)),
            pl.BlockSpec((1, 1, H), lambda i: (l, 0, 0)),
            pl.BlockSpec((1, 1, H), lambda i: (l, 0, 0)),
            pl.BlockSpec((1, 1, H), lambda i: (l, 0, 0)),
        ],
        out_specs=pl.BlockSpec((_BLK, H), lambda i: (i, 0)),
        out_shape=jax.ShapeDtypeStruct((N, H), jnp.float32),
    )(eps1, h, aggr, W1, b1.reshape(L, 1, H), W2, b2.reshape(L, 1, H),
      gam_s.reshape(L, 1, H), beta.reshape(L, 1, H))


def _head_body(batch_ref, tidx_ref, h_ref, Temb_ref, Wh1_ref, bh1_ref,
               Wh2_ref, bh2_ref, Wh3_ref, bh3_ref, out_ref, acc_ref):
    i = pl.program_id(0)
    nb = pl.num_programs(0)

    @pl.when(i == 0)
    def _init():
        acc_ref[...] = jnp.zeros_like(acc_ref)

    b = batch_ref[0, 0]
    h = h_ref[...]
    gids = lax.broadcasted_iota(jnp.int32, (G, _BLK), 0)
    onehot = (gids == b[None, :]).astype(jnp.float32)
    acc_ref[...] += jnp.dot(onehot, h, preferred_element_type=jnp.float32)

    @pl.when(i == nb - 1)
    def _fin():
        g = acc_ref[...]
        t = tidx_ref[0]
        te_oh = (lax.broadcasted_iota(jnp.int32, (G, NT), 1)
                 == t[:, None]).astype(jnp.float32)
        temb = jnp.dot(te_oh, Temb_ref[...],
                       preferred_element_type=jnp.float32)
        r1 = (jnp.dot(g, Wh1_ref[:H, :], preferred_element_type=jnp.float32)
              + jnp.dot(temb, Wh1_ref[H:, :],
                        preferred_element_type=jnp.float32)
              + bh1_ref[...])
        r1 = jnp.maximum(r1, 0.0)
        r2 = jnp.dot(r1, Wh2_ref[...], preferred_element_type=jnp.float32)
        r2 = jnp.maximum(r2 + bh2_ref[...], 0.0)
        r3 = jnp.dot(r2, Wh3_ref[...], preferred_element_type=jnp.float32)
        out_ref[...] = (r3 + bh3_ref[...]).reshape(1, G)


def _head(h, batch, target_idx, Temb, Wh1, bh1, Wh2, bh2, Wh3, bh3):
    batch3 = batch.reshape(_NB, 1, _BLK)
    tidx2 = target_idx.reshape(1, G)
    out = pl.pallas_call(
        _head_body,
        grid=(_NB,),
        in_specs=[
            pl.BlockSpec((1, 1, _BLK), lambda i: (i, 0, 0)),
            pl.BlockSpec((1, G), lambda i: (0, 0)),
            pl.BlockSpec((_BLK, H), lambda i: (i, 0)),
            pl.BlockSpec((NT, TE), lambda i: (0, 0)),
            pl.BlockSpec((H + TE, H), lambda i: (0, 0)),
            pl.BlockSpec((1, H), lambda i: (0, 0)),
            pl.BlockSpec((H, H // 2), lambda i: (0, 0)),
            pl.BlockSpec((1, H // 2), lambda i: (0, 0)),
            pl.BlockSpec((H // 2, 1), lambda i: (0, 0)),
            pl.BlockSpec((1, 1), lambda i: (0, 0)),
        ],
        out_specs=pl.BlockSpec((1, G), lambda i: (0, 0)),
        out_shape=jax.ShapeDtypeStruct((1, G), jnp.float32),
        scratch_shapes=[pltpu.VMEM((G, H), jnp.float32)],
    )(batch3, tidx2, h, Temb, Wh1, bh1.reshape(1, H), Wh2,
      bh2.reshape(1, H // 2), Wh3, bh3.reshape(1, 1))
    return out.reshape(G)


def kernel(x, edge_index, edge_attr, batch, target_idx, Wp, bp, eps, We, be,
           W1, b1, W2, b2, gamma, beta, Temb, Wh1, bh1, Wh2, bh2, Wh3, bh3):
    src = edge_index[0]
    dst = edge_index[1]
    h = _h0(x, Wp, bp)
    gam_s = gamma / jnp.sqrt(1.0 + BN_EPS)
    for l in range(L):
        ea_l = _ea_layer(edge_attr, We, be, l)
        aggr = _edge_layer(h, ea_l, src, dst)
        eps1 = eps[l].reshape(1, 1)
        h = _node(l, eps1, h, aggr, W1, b1, W2, b2, gam_s, beta)
    return _head(h, batch, target_idx, Temb, Wh1, bh1, Wh2, bh2, Wh3, bh3)
